# baseline (speedup 1.0000x reference)
import jax
import jax.numpy as jnp
from jax import lax
from jax.experimental import pallas as pl
from jax.experimental.pallas import tpu as pltpu
from jax.experimental.pallas import tpu_sc as plsc

N = 10000
E = 320000
D = 128
R = 8

NC, NS = 2, 16
CHUNK = 80
EPT = E // (NC * NS)
CPT = EPT // CHUNK
CPT_CNT = E // NS // CHUNK
CNT_PAD = 81920
NPAD = 10240
ROWS_PT = NPAD // NS
ROW_B = "rows"
IDX_B = "idx"


def _tc_weights(c1_ref, b1_ref, c2_ref, b2_ref, w1_ref, w2_ref):
    w1_ref[...] = jnp.dot(c1_ref[...], b1_ref[...], preferred_element_type=jnp.float32)
    w2_ref[...] = jnp.dot(c2_ref[...], b2_ref[...], preferred_element_type=jnp.float32)


def _tc_h(x_ref, w1_ref, w2_ref, h1_ref, h2_ref):
    x = x_ref[...]
    h1_ref[0] = jnp.dot(x, w1_ref[0], preferred_element_type=jnp.float32)
    h2_ref[0] = jnp.dot(x, w2_ref[0], preferred_element_type=jnp.float32)


def _tc_edges(src_ref, dst_ref, et_ref, ig_ref, ic_ref):
    et = et_ref[...]
    ig_ref[...] = et * N + src_ref[...]
    ic_ref[...] = et * N + dst_ref[...]


def _bcast_lane(vec, lane):
    idx = jnp.full((16,), lane, jnp.int32)[:, None]
    dn = lax.GatherDimensionNumbers(
        offset_dims=(), collapsed_slice_dims=(0,), start_index_map=(0,))
    return lax.gather(vec, idx, dn, (1,),
                      mode=lax.GatherScatterMode.PROMISE_IN_BOUNDS)


def _sc_body(ig_h, ic_h, dst_h, h1_h, h2_h, zrow_h, zblk_h,
             agg1_h, agg2_h,
             acc, winv, igb, wb, bufa, bufb, dba, dbb, ica, icb8,
             ones_v, dm80, tmp_v,
             sem_la, sem_lb, sem_fa, sem_fb, sem_ga, sem_gb, sem_sa, sem_sb,
             sem_da, sem_db):
    c = lax.axis_index("c")
    s = lax.axis_index("s")
    tid = c * NS + s

    SB = 2000

    def stage_main(t, carry):
        pltpu.sync_copy(ig_h.at[pl.ds(tid * EPT + t * SB, SB)],
                        igb.at[pl.ds(t * SB, SB)])
        return carry

    lax.fori_loop(0, EPT // SB, stage_main, 0)

    def stage_zero(t, carry):
        pltpu.sync_copy(zrow_h, winv.at[pl.ds(s * 5120 + t * 640, 640)])
        return carry

    lax.fori_loop(0, 8, stage_zero, 0)
    pltpu.sync_copy(zblk_h, acc.at[pl.ds(s * ROWS_PT, ROWS_PT)])
    for g in range(CHUNK // 16):
        ones_v[pl.ds(g * 16, 16)] = jnp.full((16,), 1.0, jnp.float32)
    plsc.subcore_barrier()

    def drain(sem, tag):
        if tag == IDX_B:
            pltpu.make_async_copy(zrow_h.at[pl.ds(0, CHUNK)], dm80, sem).wait()
        else:
            pltpu.make_async_copy(h1_h.at[pl.ds(0, CHUNK)], bufa, sem).wait()

    NCC = (E // NS) // CHUNK
    cbase = s * (E // NS)

    def cld(i, buf, sem):
        pltpu.async_copy(ic_h.at[pl.ds(cbase + i * CHUNK, CHUNK)], buf, sem)

    cld(0, ica, sem_la)

    def cnt_body(t, carry):
        i0 = 2 * t
        i2 = jnp.minimum(i0 + 2, NCC - 1)
        drain(sem_la, IDX_B)
        cld(i0 + 1, icb8, sem_lb)
        pltpu.async_copy(ones_v, winv.at[ica], sem_fa, add=True)
        drain(sem_lb, IDX_B)
        drain(sem_fa, IDX_B)
        cld(i2, ica, sem_la)
        pltpu.async_copy(ones_v, winv.at[icb8], sem_fb, add=True)
        drain(sem_fb, IDX_B)
        return carry

    lax.fori_loop(0, NCC // 2, cnt_body, 0)
    drain(sem_la, IDX_B)
    plsc.subcore_barrier()

    def inv_blk(t, carry):
        pltpu.sync_copy(winv.at[pl.ds(s * 5120 + t * 640, 640)], tmp_v)

        def inv_body(g, carry2):
            v = tmp_v[pl.ds(g * 16, 16)]
            tmp_v[pl.ds(g * 16, 16)] = 1.0 / jnp.maximum(v, 1.0)
            return carry2

        lax.fori_loop(0, 640 // 16, inv_body, 0)
        pltpu.sync_copy(tmp_v, winv.at[pl.ds(s * 5120 + t * 640, 640)])
        return carry

    lax.fori_loop(0, 8, inv_blk, 0)
    plsc.subcore_barrier()

    def wld(i, buf, sem):
        pltpu.async_copy(ic_h.at[pl.ds(tid * EPT + i * CHUNK, CHUNK)], buf, sem)

    wld(0, ica, sem_la)

    def wg_body(t, carry):
        i0 = 2 * t
        drain(sem_la, IDX_B)
        wld(i0 + 1, icb8, sem_lb)
        pltpu.async_copy(winv.at[ica], wb.at[pl.ds(i0 * CHUNK, CHUNK)], sem_fa)
        drain(sem_lb, IDX_B)
        drain(sem_fa, IDX_B)
        wld(i0 + 2, ica, sem_la)
        pltpu.async_copy(winv.at[icb8], wb.at[pl.ds((i0 + 1) * CHUNK, CHUNK)],
                         sem_fb)
        drain(sem_fb, IDX_B)
        return carry

    lax.fori_loop(0, (CPT - 1) // 2, wg_body, 0)
    drain(sem_la, IDX_B)
    pltpu.async_copy(winv.at[ica], wb.at[pl.ds((CPT - 1) * CHUNK, CHUNK)],
                     sem_fa)
    drain(sem_fa, IDX_B)

    def scale(buf, ci):
        def grp(g, carry):
            wg = wb[pl.ds(ci * CHUNK + g * 16, 16)]
            for l in range(16):
                w1 = _bcast_lane(wg, l)
                j = g * 16 + l
                for k in range(D // 16):
                    buf[j, pl.ds(k * 16, 16)] = buf[j, pl.ds(k * 16, 16)] * w1
            return carry

        lax.fori_loop(0, CHUNK // 16, grp, 0)

    def do_layer(h_h, agg_h):
        def gidx(i):
            return igb.at[pl.ds(i * CHUNK, CHUNK)]

        def dld(i, buf, sem):
            pltpu.async_copy(dst_h.at[pl.ds(tid * EPT + i * CHUNK, CHUNK)],
                             buf, sem)

        dld(0, dba, sem_da)
        pltpu.async_copy(h_h.at[gidx(0)], bufa, sem_ga)

        def pair_body(t, carry):
            i0 = 2 * t

            @pl.when(t > 0)
            def _():
                drain(sem_sb, ROW_B)
            dld(i0 + 1, dbb, sem_db)
            drain(sem_ga, ROW_B)
            pltpu.async_copy(h_h.at[gidx(i0 + 1)], bufb, sem_gb)
            drain(sem_da, IDX_B)
            scale(bufa, i0)
            pltpu.async_copy(bufa, acc.at[dba], sem_sa, add=True)
            drain(sem_gb, ROW_B)
            drain(sem_sa, ROW_B)
            dld(i0 + 2, dba, sem_da)
            pltpu.async_copy(h_h.at[gidx(i0 + 2)], bufa, sem_ga)
            drain(sem_db, IDX_B)
            scale(bufb, i0 + 1)
            pltpu.async_copy(bufb, acc.at[dbb], sem_sb, add=True)
            return carry

        lax.fori_loop(0, (CPT - 1) // 2, pair_body, 0)
        drain(sem_sb, ROW_B)
        drain(sem_ga, ROW_B)
        drain(sem_da, IDX_B)
        scale(bufa, CPT - 1)
        pltpu.async_copy(bufa, acc.at[dba], sem_sa, add=True)
        drain(sem_sa, ROW_B)
        plsc.subcore_barrier()
        pltpu.sync_copy(acc.at[pl.ds(s * ROWS_PT, ROWS_PT)],
                        agg_h.at[c, pl.ds(s * ROWS_PT, ROWS_PT)])
        plsc.subcore_barrier()

    do_layer(h1_h, agg1_h)
    pltpu.sync_copy(zblk_h, acc.at[pl.ds(s * ROWS_PT, ROWS_PT)])
    plsc.subcore_barrier()
    do_layer(h2_h, agg2_h)


def _tc_final(bs_ref, xt_ref, a1_ref, a2_ref, r1_ref, b1_ref, r2_ref, b2_ref,
              pa_ref, out_ref):
    i = pl.program_id(0)
    rows = xt_ref.shape[0]
    a = pa_ref[...]
    h1 = (jnp.dot(xt_ref[...], r1_ref[...], preferred_element_type=jnp.float32)
          + b1_ref[...] + a1_ref[0] + a1_ref[1])
    h1 = jnp.where(h1 >= 0, h1, h1 * a)
    ridx = i * rows + lax.broadcasted_iota(jnp.int32, (rows, D), 0)
    h1 = jnp.where(ridx < bs_ref[0], h1, 0.0)
    h2 = (jnp.dot(h1, r2_ref[...], preferred_element_type=jnp.float32)
          + b2_ref[...] + a2_ref[0] + a2_ref[1])
    out_ref[...] = jnp.where(h2 >= 0, h2, h2 * a)


def kernel(x_src, x_target, edge_index, edge_type, batch_size,
           comp1, basis1, root1, bias1, comp2, basis2, root2, bias2, prelu_a):
    f32 = jnp.float32

    b1f = basis1.reshape(16, D * D)
    b2f = basis2.reshape(16, D * D)
    w1f, w2f = pl.pallas_call(
        _tc_weights,
        out_shape=[jax.ShapeDtypeStruct((R, D * D), f32)] * 2,
    )(comp1, b1f, comp2, b2f)
    w1 = w1f.reshape(R, D, D)
    w2 = w2f.reshape(R, D, D)

    nb = 5
    rows = N // nb
    h1, h2 = pl.pallas_call(
        _tc_h,
        grid=(R, nb),
        in_specs=[
            pl.BlockSpec((rows, D), lambda r, b: (b, 0)),
            pl.BlockSpec((1, D, D), lambda r, b: (r, 0, 0)),
            pl.BlockSpec((1, D, D), lambda r, b: (r, 0, 0)),
        ],
        out_specs=[
            pl.BlockSpec((1, rows, D), lambda r, b: (r, b, 0)),
            pl.BlockSpec((1, rows, D), lambda r, b: (r, b, 0)),
        ],
        out_shape=[jax.ShapeDtypeStruct((R, N, D), f32)] * 2,
    )(x_src, w1, w2)
    h1 = h1.reshape(R * N, D)
    h2 = h2.reshape(R * N, D)

    src2 = edge_index[0].reshape(E // D, D)
    dst2 = edge_index[1].reshape(E // D, D)
    et2 = edge_type.reshape(E // D, D)
    ig2, ic2 = pl.pallas_call(
        _tc_edges,
        out_shape=[jax.ShapeDtypeStruct((E // D, D), jnp.int32)] * 2,
    )(src2, dst2, et2)
    ig = ig2.reshape(E)
    ic = ic2.reshape(E)
    dst = edge_index[1]

    mesh = plsc.VectorSubcoreMesh(core_axis_name="c", subcore_axis_name="s")
    sc = pl.kernel(
        _sc_body,
        mesh=mesh,
        out_type=[jax.ShapeDtypeStruct((NC, NPAD, D), f32)] * 2,
        scratch_types=[
            pltpu.VMEM_SHARED((NPAD, D), f32),
            pltpu.VMEM_SHARED((CNT_PAD,), f32),
            pltpu.VMEM((EPT,), jnp.int32),
            pltpu.VMEM((EPT,), f32),
            pltpu.VMEM((CHUNK, D), f32),
            pltpu.VMEM((CHUNK, D), f32),
            pltpu.VMEM((CHUNK,), jnp.int32),
            pltpu.VMEM((CHUNK,), jnp.int32),
            pltpu.VMEM((CHUNK,), jnp.int32),
            pltpu.VMEM((CHUNK,), jnp.int32),
            pltpu.VMEM((CHUNK,), f32),
            pltpu.VMEM((CHUNK,), f32),
            pltpu.VMEM((640,), f32),
        ] + [pltpu.SemaphoreType.DMA] * 10,
    )
    zrow = jnp.zeros((640,), f32)
    zblk = jnp.zeros((ROWS_PT, D), f32)
    agg1p, agg2p = sc(ig, ic, dst, h1, h2, zrow, zblk)

    bs = jnp.asarray(batch_size, jnp.int32).reshape(1)
    out = pl.pallas_call(
        _tc_final,
        grid=(nb,),
        in_specs=[
            pl.BlockSpec(memory_space=pltpu.SMEM),
            pl.BlockSpec((rows, D), lambda i: (i, 0)),
            pl.BlockSpec((NC, rows, D), lambda i: (0, i, 0)),
            pl.BlockSpec((NC, rows, D), lambda i: (0, i, 0)),
            pl.BlockSpec((D, D), lambda i: (0, 0)),
            pl.BlockSpec((1, D), lambda i: (0, 0)),
            pl.BlockSpec((D, D), lambda i: (0, 0)),
            pl.BlockSpec((1, D), lambda i: (0, 0)),
            pl.BlockSpec((1, D), lambda i: (0, 0)),
        ],
        out_specs=pl.BlockSpec((rows, D), lambda i: (i, 0)),
        out_shape=jax.ShapeDtypeStruct((N, D), f32),
    )(bs, x_target, agg1p, agg2p, root1, bias1.reshape(1, D),
      root2, bias2.reshape(1, D), prelu_a.reshape(1, D))
    return out


# speedup vs baseline: 1.2132x; 1.2132x over previous
import jax
import jax.numpy as jnp
from jax import lax
from jax.experimental import pallas as pl
from jax.experimental.pallas import tpu as pltpu
from jax.experimental.pallas import tpu_sc as plsc

N = 10000
E = 320000
D = 128
R = 8

NC, NS = 2, 16
CHUNK = 80
EPT = E // (NC * NS)
CPT = EPT // CHUNK
CPT_CNT = E // NS // CHUNK
CNT_PAD = 81920
NPAD = 10240
ROWS_PT = NPAD // NS
ROW_B = "rows"
IDX_B = "idx"


def _tc_weights(c1_ref, b1_ref, c2_ref, b2_ref, w1_ref, w2_ref):
    w1_ref[...] = jnp.dot(c1_ref[...], b1_ref[...], preferred_element_type=jnp.float32)
    w2_ref[...] = jnp.dot(c2_ref[...], b2_ref[...], preferred_element_type=jnp.float32)


def _tc_h(x_ref, w1_ref, w2_ref, h1_ref, h2_ref):
    x = x_ref[...]
    h1_ref[0] = jnp.dot(x, w1_ref[0], preferred_element_type=jnp.float32)
    h2_ref[0] = jnp.dot(x, w2_ref[0], preferred_element_type=jnp.float32)


def _tc_edges(src_ref, dst_ref, et_ref, ig_ref, ic_ref):
    et = et_ref[...]
    ig_ref[...] = et * N + src_ref[...]
    ic_ref[...] = et * N + dst_ref[...]


def _bcast_lane(vec, lane):
    idx = jnp.full((16,), lane, jnp.int32)[:, None]
    dn = lax.GatherDimensionNumbers(
        offset_dims=(), collapsed_slice_dims=(0,), start_index_map=(0,))
    return lax.gather(vec, idx, dn, (1,),
                      mode=lax.GatherScatterMode.PROMISE_IN_BOUNDS)


def _sc_body(ig_h, ic_h, dst_h, h1_h, h2_h, zrow_h, zblk_h,
             agg1_h, agg2_h,
             acc, winv, igb, wb, bufa, bufb, dba, dbb, ica, icb8,
             ones_v, dm80, tmp_v,
             sem_la, sem_lb, sem_fa, sem_fb, sem_ga, sem_gb, sem_sa, sem_sb,
             sem_da, sem_db):
    c = lax.axis_index("c")
    s = lax.axis_index("s")
    tid = c * NS + s

    SB = 2000

    def stage_main(t, carry):
        pltpu.sync_copy(ig_h.at[pl.ds(tid * EPT + t * SB, SB)],
                        igb.at[pl.ds(t * SB, SB)])
        return carry

    lax.fori_loop(0, EPT // SB, stage_main, 0)

    def stage_zero(t, carry):
        pltpu.sync_copy(zrow_h, winv.at[pl.ds(s * 5120 + t * 640, 640)])
        return carry

    lax.fori_loop(0, 8, stage_zero, 0)
    pltpu.sync_copy(zblk_h, acc.at[pl.ds(s * ROWS_PT, ROWS_PT)])
    for g in range(CHUNK // 16):
        ones_v[pl.ds(g * 16, 16)] = jnp.full((16,), 1.0, jnp.float32)
    plsc.subcore_barrier()

    def drain(sem, tag):
        if tag == IDX_B:
            pltpu.make_async_copy(zrow_h.at[pl.ds(0, CHUNK)], dm80, sem).wait()
        else:
            pltpu.make_async_copy(h1_h.at[pl.ds(0, CHUNK)], bufa, sem).wait()

    NCC = (E // NS) // CHUNK
    cbase = s * (E // NS)

    def cld(i, buf, sem):
        pltpu.async_copy(ic_h.at[pl.ds(cbase + i * CHUNK, CHUNK)], buf, sem)

    LD4 = [sem_la, sem_lb, sem_da, sem_db]
    SC4 = [sem_fa, sem_fb, sem_sa, sem_sb]
    IB4 = [ica, icb8, dba, dbb]

    def cnt_body(t, carry):
        q0 = 4 * t
        for k in range(4):
            @pl.when(t > 0)
            def _(k=k):
                drain(SC4[k], IDX_B)

            cld(q0 + k, IB4[k], LD4[k])
        for k in range(4):
            drain(LD4[k], IDX_B)
            pltpu.async_copy(ones_v, winv.at[IB4[k]], SC4[k], add=True)
        return carry

    lax.fori_loop(0, NCC // 4, cnt_body, 0)
    for k in range(2):
        drain(SC4[k], IDX_B)
        cld(NCC - 2 + k, IB4[k], LD4[k])
    for k in range(2, 4):
        drain(SC4[k], IDX_B)
    for k in range(2):
        drain(LD4[k], IDX_B)
        pltpu.async_copy(ones_v, winv.at[IB4[k]], SC4[k], add=True)
    for k in range(2):
        drain(SC4[k], IDX_B)
    plsc.subcore_barrier()

    def inv_blk(t, carry):
        pltpu.sync_copy(winv.at[pl.ds(s * 5120 + t * 640, 640)], tmp_v)

        def inv_body(g, carry2):
            v = tmp_v[pl.ds(g * 16, 16)]
            tmp_v[pl.ds(g * 16, 16)] = 1.0 / jnp.maximum(v, 1.0)
            return carry2

        lax.fori_loop(0, 640 // 16, inv_body, 0)
        pltpu.sync_copy(tmp_v, winv.at[pl.ds(s * 5120 + t * 640, 640)])
        return carry

    lax.fori_loop(0, 8, inv_blk, 0)
    plsc.subcore_barrier()

    NWC = EPT // CHUNK

    def wld(i, buf, sem):
        pltpu.async_copy(ic_h.at[pl.ds(tid * EPT + i * CHUNK, CHUNK)], buf, sem)

    def wg_body(t, carry):
        q0 = 4 * t
        for k in range(4):
            @pl.when(t > 0)
            def _(k=k):
                drain(SC4[k], IDX_B)

            wld(q0 + k, IB4[k], LD4[k])
        for k in range(4):
            drain(LD4[k], IDX_B)
            pltpu.async_copy(winv.at[IB4[k]], wb.at[pl.ds((q0 + k) * CHUNK, CHUNK)],
                             SC4[k])
        return carry

    lax.fori_loop(0, NWC // 4, wg_body, 0)
    drain(SC4[0], IDX_B)
    wld(NWC - 1, IB4[0], LD4[0])
    for k in range(1, 4):
        drain(SC4[k], IDX_B)
    drain(LD4[0], IDX_B)
    pltpu.async_copy(winv.at[IB4[0]], wb.at[pl.ds((NWC - 1) * CHUNK, CHUNK)],
                     SC4[0])
    drain(SC4[0], IDX_B)

    def scale(buf, ci):
        def grp(g, carry):
            wg = wb[pl.ds(ci * CHUNK + g * 16, 16)]
            for l in range(16):
                w1 = _bcast_lane(wg, l)
                j = g * 16 + l
                for k in range(D // 16):
                    buf[j, pl.ds(k * 16, 16)] = buf[j, pl.ds(k * 16, 16)] * w1
            return carry

        lax.fori_loop(0, CHUNK // 16, grp, 0)

    def do_layer(h_h, agg_h):
        def gidx(i):
            return igb.at[pl.ds(i * CHUNK, CHUNK)]

        def dld(i, buf, sem):
            pltpu.async_copy(dst_h.at[pl.ds(tid * EPT + i * CHUNK, CHUNK)],
                             buf, sem)

        dld(0, dba, sem_da)
        pltpu.async_copy(h_h.at[gidx(0)], bufa, sem_ga)

        def pair_body(t, carry):
            i0 = 2 * t

            @pl.when(t > 0)
            def _():
                drain(sem_sb, ROW_B)
            dld(i0 + 1, dbb, sem_db)
            drain(sem_ga, ROW_B)
            pltpu.async_copy(h_h.at[gidx(i0 + 1)], bufb, sem_gb)
            drain(sem_da, IDX_B)
            scale(bufa, i0)
            pltpu.async_copy(bufa, acc.at[dba], sem_sa, add=True)
            drain(sem_gb, ROW_B)
            drain(sem_sa, ROW_B)
            dld(i0 + 2, dba, sem_da)
            pltpu.async_copy(h_h.at[gidx(i0 + 2)], bufa, sem_ga)
            drain(sem_db, IDX_B)
            scale(bufb, i0 + 1)
            pltpu.async_copy(bufb, acc.at[dbb], sem_sb, add=True)
            return carry

        lax.fori_loop(0, (CPT - 1) // 2, pair_body, 0)
        drain(sem_sb, ROW_B)
        drain(sem_ga, ROW_B)
        drain(sem_da, IDX_B)
        scale(bufa, CPT - 1)
        pltpu.async_copy(bufa, acc.at[dba], sem_sa, add=True)
        drain(sem_sa, ROW_B)
        plsc.subcore_barrier()
        pltpu.sync_copy(acc.at[pl.ds(s * ROWS_PT, ROWS_PT)],
                        agg_h.at[c, pl.ds(s * ROWS_PT, ROWS_PT)])
        plsc.subcore_barrier()

    do_layer(h1_h, agg1_h)
    pltpu.sync_copy(zblk_h, acc.at[pl.ds(s * ROWS_PT, ROWS_PT)])
    plsc.subcore_barrier()
    do_layer(h2_h, agg2_h)


def _tc_final(bs_ref, xt_ref, a1_ref, a2_ref, r1_ref, b1_ref, r2_ref, b2_ref,
              pa_ref, out_ref):
    i = pl.program_id(0)
    rows = xt_ref.shape[0]
    a = pa_ref[...]
    h1 = (jnp.dot(xt_ref[...], r1_ref[...], preferred_element_type=jnp.float32)
          + b1_ref[...] + a1_ref[0] + a1_ref[1])
    h1 = jnp.where(h1 >= 0, h1, h1 * a)
    ridx = i * rows + lax.broadcasted_iota(jnp.int32, (rows, D), 0)
    h1 = jnp.where(ridx < bs_ref[0], h1, 0.0)
    h2 = (jnp.dot(h1, r2_ref[...], preferred_element_type=jnp.float32)
          + b2_ref[...] + a2_ref[0] + a2_ref[1])
    out_ref[...] = jnp.where(h2 >= 0, h2, h2 * a)


def kernel(x_src, x_target, edge_index, edge_type, batch_size,
           comp1, basis1, root1, bias1, comp2, basis2, root2, bias2, prelu_a):
    f32 = jnp.float32

    b1f = basis1.reshape(16, D * D)
    b2f = basis2.reshape(16, D * D)
    w1f, w2f = pl.pallas_call(
        _tc_weights,
        out_shape=[jax.ShapeDtypeStruct((R, D * D), f32)] * 2,
    )(comp1, b1f, comp2, b2f)
    w1 = w1f.reshape(R, D, D)
    w2 = w2f.reshape(R, D, D)

    nb = 5
    rows = N // nb
    h1, h2 = pl.pallas_call(
        _tc_h,
        grid=(R, nb),
        in_specs=[
            pl.BlockSpec((rows, D), lambda r, b: (b, 0)),
            pl.BlockSpec((1, D, D), lambda r, b: (r, 0, 0)),
            pl.BlockSpec((1, D, D), lambda r, b: (r, 0, 0)),
        ],
        out_specs=[
            pl.BlockSpec((1, rows, D), lambda r, b: (r, b, 0)),
            pl.BlockSpec((1, rows, D), lambda r, b: (r, b, 0)),
        ],
        out_shape=[jax.ShapeDtypeStruct((R, N, D), f32)] * 2,
    )(x_src, w1, w2)
    h1 = h1.reshape(R * N, D)
    h2 = h2.reshape(R * N, D)

    src2 = edge_index[0].reshape(E // D, D)
    dst2 = edge_index[1].reshape(E // D, D)
    et2 = edge_type.reshape(E // D, D)
    ig2, ic2 = pl.pallas_call(
        _tc_edges,
        out_shape=[jax.ShapeDtypeStruct((E // D, D), jnp.int32)] * 2,
    )(src2, dst2, et2)
    ig = ig2.reshape(E)
    ic = ic2.reshape(E)
    dst = edge_index[1]

    mesh = plsc.VectorSubcoreMesh(core_axis_name="c", subcore_axis_name="s")
    sc = pl.kernel(
        _sc_body,
        mesh=mesh,
        out_type=[jax.ShapeDtypeStruct((NC, NPAD, D), f32)] * 2,
        scratch_types=[
            pltpu.VMEM_SHARED((NPAD, D), f32),
            pltpu.VMEM_SHARED((CNT_PAD,), f32),
            pltpu.VMEM((EPT,), jnp.int32),
            pltpu.VMEM((EPT,), f32),
            pltpu.VMEM((CHUNK, D), f32),
            pltpu.VMEM((CHUNK, D), f32),
            pltpu.VMEM((CHUNK,), jnp.int32),
            pltpu.VMEM((CHUNK,), jnp.int32),
            pltpu.VMEM((CHUNK,), jnp.int32),
            pltpu.VMEM((CHUNK,), jnp.int32),
            pltpu.VMEM((CHUNK,), f32),
            pltpu.VMEM((CHUNK,), f32),
            pltpu.VMEM((640,), f32),
        ] + [pltpu.SemaphoreType.DMA] * 10,
    )
    zrow = jnp.zeros((640,), f32)
    zblk = jnp.zeros((ROWS_PT, D), f32)
    agg1p, agg2p = sc(ig, ic, dst, h1, h2, zrow, zblk)

    bs = jnp.asarray(batch_size, jnp.int32).reshape(1)
    out = pl.pallas_call(
        _tc_final,
        grid=(nb,),
        in_specs=[
            pl.BlockSpec(memory_space=pltpu.SMEM),
            pl.BlockSpec((rows, D), lambda i: (i, 0)),
            pl.BlockSpec((NC, rows, D), lambda i: (0, i, 0)),
            pl.BlockSpec((NC, rows, D), lambda i: (0, i, 0)),
            pl.BlockSpec((D, D), lambda i: (0, 0)),
            pl.BlockSpec((1, D), lambda i: (0, 0)),
            pl.BlockSpec((D, D), lambda i: (0, 0)),
            pl.BlockSpec((1, D), lambda i: (0, 0)),
            pl.BlockSpec((1, D), lambda i: (0, 0)),
        ],
        out_specs=pl.BlockSpec((rows, D), lambda i: (i, 0)),
        out_shape=jax.ShapeDtypeStruct((N, D), f32),
    )(bs, x_target, agg1p, agg2p, root1, bias1.reshape(1, D),
      root2, bias2.reshape(1, D), prelu_a.reshape(1, D))
    return out


# speedup vs baseline: 1.2995x; 1.0711x over previous
import jax
import jax.numpy as jnp
from jax import lax
from jax.experimental import pallas as pl
from jax.experimental.pallas import tpu as pltpu
from jax.experimental.pallas import tpu_sc as plsc

N = 10000
E = 320000
D = 128
R = 8

NC, NS = 2, 16
CHUNK = 80
EPT = E // (NC * NS)
CPT = EPT // CHUNK
CPT_CNT = E // NS // CHUNK
CNT_PAD = 81920
NPAD = 10240
ROWS_PT = NPAD // NS
ROW_B = "rows"
IDX_B = "idx"


def _tc_weights(c1_ref, b1_ref, c2_ref, b2_ref, w1_ref, w2_ref):
    w1_ref[...] = jnp.dot(c1_ref[...], b1_ref[...], preferred_element_type=jnp.float32)
    w2_ref[...] = jnp.dot(c2_ref[...], b2_ref[...], preferred_element_type=jnp.float32)


def _tc_h(x_ref, w1_ref, w2_ref, h1_ref, h2_ref):
    x = x_ref[...]
    h1_ref[0] = jnp.dot(x, w1_ref[0], preferred_element_type=jnp.float32)
    h2_ref[0] = jnp.dot(x, w2_ref[0], preferred_element_type=jnp.float32)


def _tc_edges(src_ref, dst_ref, et_ref, ig_ref, ic_ref):
    et = et_ref[...]
    ig_ref[...] = et * N + src_ref[...]
    ic_ref[...] = et * N + dst_ref[...]


def _bcast_lane(vec, lane):
    idx = jnp.full((16,), lane, jnp.int32)[:, None]
    dn = lax.GatherDimensionNumbers(
        offset_dims=(), collapsed_slice_dims=(0,), start_index_map=(0,))
    return lax.gather(vec, idx, dn, (1,),
                      mode=lax.GatherScatterMode.PROMISE_IN_BOUNDS)


def _sc_body(ig_h, ic_h, dst_h, h1_h, h2_h, zrow_h, zblk_h,
             agg1_h, agg2_h,
             acc, winv, igb, wb, bufa, bufb, dba, dbb, ica, icb8,
             ones_v, dm80, tmp_v,
             sem_la, sem_lb, sem_fa, sem_fb, sem_ga, sem_gb, sem_sa, sem_sb,
             sem_da, sem_db, sem_st1, sem_st2):
    c = lax.axis_index("c")
    s = lax.axis_index("s")
    tid = c * NS + s

    SB = 2000

    def stage_main(t, carry):
        pltpu.async_copy(ig_h.at[pl.ds(tid * EPT + t * SB, SB)],
                         igb.at[pl.ds(t * SB, SB)], sem_st1)
        return carry

    lax.fori_loop(0, EPT // SB, stage_main, 0)

    def stage_zero(t, carry):
        pltpu.sync_copy(zrow_h, winv.at[pl.ds(s * 5120 + t * 640, 640)])
        return carry

    lax.fori_loop(0, 8, stage_zero, 0)
    pltpu.async_copy(zblk_h, acc.at[pl.ds(s * ROWS_PT, ROWS_PT)], sem_st2)
    for g in range(CHUNK // 16):
        ones_v[pl.ds(g * 16, 16)] = jnp.full((16,), 1.0, jnp.float32)
    plsc.subcore_barrier()

    def drain(sem, tag):
        if tag == IDX_B:
            pltpu.make_async_copy(zrow_h.at[pl.ds(0, CHUNK)], dm80, sem).wait()
        else:
            pltpu.make_async_copy(h1_h.at[pl.ds(0, CHUNK)], bufa, sem).wait()

    NCC = (E // NS) // CHUNK
    cbase = s * (E // NS)

    def cld(i, buf, sem):
        pltpu.async_copy(ic_h.at[pl.ds(cbase + i * CHUNK, CHUNK)], buf, sem)

    LD4 = [sem_la, sem_lb, sem_da, sem_db]
    SC4 = [sem_fa, sem_fb, sem_sa, sem_sb]
    IB4 = [ica, icb8, dba, dbb]

    def cnt_body(t, carry):
        q0 = 4 * t
        for k in range(4):
            @pl.when(t > 0)
            def _(k=k):
                drain(SC4[k], IDX_B)

            cld(q0 + k, IB4[k], LD4[k])
        for k in range(4):
            drain(LD4[k], IDX_B)
            pltpu.async_copy(ones_v, winv.at[IB4[k]], SC4[k], add=True)
        return carry

    lax.fori_loop(0, NCC // 4, cnt_body, 0)
    for k in range(2):
        drain(SC4[k], IDX_B)
        cld(NCC - 2 + k, IB4[k], LD4[k])
    for k in range(2, 4):
        drain(SC4[k], IDX_B)
    for k in range(2):
        drain(LD4[k], IDX_B)
        pltpu.async_copy(ones_v, winv.at[IB4[k]], SC4[k], add=True)
    for k in range(2):
        drain(SC4[k], IDX_B)
    plsc.subcore_barrier()

    def inv_blk(t, carry):
        pltpu.sync_copy(winv.at[pl.ds(s * 5120 + t * 640, 640)], tmp_v)

        def inv_body(g, carry2):
            v = tmp_v[pl.ds(g * 16, 16)]
            tmp_v[pl.ds(g * 16, 16)] = 1.0 / jnp.maximum(v, 1.0)
            return carry2

        lax.fori_loop(0, 640 // 16, inv_body, 0)
        pltpu.sync_copy(tmp_v, winv.at[pl.ds(s * 5120 + t * 640, 640)])
        return carry

    lax.fori_loop(0, 8, inv_blk, 0)

    def stage_drain(t, carry):
        pltpu.make_async_copy(ig_h.at[pl.ds(0, SB)],
                              igb.at[pl.ds(0, SB)], sem_st1).wait()
        return carry

    lax.fori_loop(0, EPT // SB, stage_drain, 0)
    pltpu.make_async_copy(zblk_h, acc.at[pl.ds(0, ROWS_PT)], sem_st2).wait()
    plsc.subcore_barrier()

    NWC = EPT // CHUNK

    def wld(i, buf, sem):
        pltpu.async_copy(ic_h.at[pl.ds(tid * EPT + i * CHUNK, CHUNK)], buf, sem)

    def wg_body(t, carry):
        q0 = 4 * t
        for k in range(4):
            @pl.when(t > 0)
            def _(k=k):
                drain(SC4[k], IDX_B)

            wld(q0 + k, IB4[k], LD4[k])
        for k in range(4):
            drain(LD4[k], IDX_B)
            pltpu.async_copy(winv.at[IB4[k]], wb.at[pl.ds((q0 + k) * CHUNK, CHUNK)],
                             SC4[k])
        return carry

    lax.fori_loop(0, NWC // 4, wg_body, 0)
    drain(SC4[0], IDX_B)
    wld(NWC - 1, IB4[0], LD4[0])
    for k in range(1, 4):
        drain(SC4[k], IDX_B)
    drain(LD4[0], IDX_B)
    pltpu.async_copy(winv.at[IB4[0]], wb.at[pl.ds((NWC - 1) * CHUNK, CHUNK)],
                     SC4[0])
    drain(SC4[0], IDX_B)

    def scale(buf, ci):
        def grp(g, carry):
            wg = wb[pl.ds(ci * CHUNK + g * 16, 16)]
            for l in range(16):
                w1 = _bcast_lane(wg, l)
                j = g * 16 + l
                for k in range(D // 16):
                    buf[j, pl.ds(k * 16, 16)] = buf[j, pl.ds(k * 16, 16)] * w1
            return carry

        lax.fori_loop(0, CHUNK // 16, grp, 0)

    def do_layer(h_h, agg_h):
        def gidx(i):
            return igb.at[pl.ds(i * CHUNK, CHUNK)]

        def dld(i, buf, sem):
            pltpu.async_copy(dst_h.at[pl.ds(tid * EPT + i * CHUNK, CHUNK)],
                             buf, sem)

        dld(0, dba, sem_da)
        pltpu.async_copy(h_h.at[gidx(0)], bufa, sem_ga)

        def pair_body(t, carry):
            i0 = 2 * t

            @pl.when(t > 0)
            def _():
                drain(sem_sb, ROW_B)
            dld(i0 + 1, dbb, sem_db)
            pltpu.async_copy(h_h.at[gidx(i0 + 1)], bufb, sem_gb)
            drain(sem_ga, ROW_B)
            drain(sem_da, IDX_B)
            scale(bufa, i0)
            pltpu.async_copy(bufa, acc.at[dba], sem_sa, add=True)
            drain(sem_gb, ROW_B)
            drain(sem_sa, ROW_B)
            dld(i0 + 2, dba, sem_da)
            pltpu.async_copy(h_h.at[gidx(i0 + 2)], bufa, sem_ga)
            drain(sem_db, IDX_B)
            scale(bufb, i0 + 1)
            pltpu.async_copy(bufb, acc.at[dbb], sem_sb, add=True)
            return carry

        lax.fori_loop(0, (CPT - 1) // 2, pair_body, 0)
        drain(sem_sb, ROW_B)
        drain(sem_ga, ROW_B)
        drain(sem_da, IDX_B)
        scale(bufa, CPT - 1)
        pltpu.async_copy(bufa, acc.at[dba], sem_sa, add=True)
        drain(sem_sa, ROW_B)
        plsc.subcore_barrier()
        pltpu.sync_copy(acc.at[pl.ds(s * ROWS_PT, ROWS_PT)],
                        agg_h.at[c, pl.ds(s * ROWS_PT, ROWS_PT)])
        plsc.subcore_barrier()

    do_layer(h1_h, agg1_h)
    pltpu.sync_copy(zblk_h, acc.at[pl.ds(s * ROWS_PT, ROWS_PT)])
    plsc.subcore_barrier()
    do_layer(h2_h, agg2_h)


def _tc_final(bs_ref, xt_ref, a1_ref, a2_ref, r1_ref, b1_ref, r2_ref, b2_ref,
              pa_ref, out_ref):
    i = pl.program_id(0)
    rows = xt_ref.shape[0]
    a = pa_ref[...]
    h1 = (jnp.dot(xt_ref[...], r1_ref[...], preferred_element_type=jnp.float32)
          + b1_ref[...] + a1_ref[0] + a1_ref[1])
    h1 = jnp.where(h1 >= 0, h1, h1 * a)
    ridx = i * rows + lax.broadcasted_iota(jnp.int32, (rows, D), 0)
    h1 = jnp.where(ridx < bs_ref[0], h1, 0.0)
    h2 = (jnp.dot(h1, r2_ref[...], preferred_element_type=jnp.float32)
          + b2_ref[...] + a2_ref[0] + a2_ref[1])
    out_ref[...] = jnp.where(h2 >= 0, h2, h2 * a)


def kernel(x_src, x_target, edge_index, edge_type, batch_size,
           comp1, basis1, root1, bias1, comp2, basis2, root2, bias2, prelu_a):
    f32 = jnp.float32

    b1f = basis1.reshape(16, D * D)
    b2f = basis2.reshape(16, D * D)
    w1f, w2f = pl.pallas_call(
        _tc_weights,
        out_shape=[jax.ShapeDtypeStruct((R, D * D), f32)] * 2,
    )(comp1, b1f, comp2, b2f)
    w1 = w1f.reshape(R, D, D)
    w2 = w2f.reshape(R, D, D)

    nb = 5
    rows = N // nb
    h1, h2 = pl.pallas_call(
        _tc_h,
        grid=(R, nb),
        in_specs=[
            pl.BlockSpec((rows, D), lambda r, b: (b, 0)),
            pl.BlockSpec((1, D, D), lambda r, b: (r, 0, 0)),
            pl.BlockSpec((1, D, D), lambda r, b: (r, 0, 0)),
        ],
        out_specs=[
            pl.BlockSpec((1, rows, D), lambda r, b: (r, b, 0)),
            pl.BlockSpec((1, rows, D), lambda r, b: (r, b, 0)),
        ],
        out_shape=[jax.ShapeDtypeStruct((R, N, D), f32)] * 2,
    )(x_src, w1, w2)
    h1 = h1.reshape(R * N, D)
    h2 = h2.reshape(R * N, D)

    src2 = edge_index[0].reshape(E // D, D)
    dst2 = edge_index[1].reshape(E // D, D)
    et2 = edge_type.reshape(E // D, D)
    ig2, ic2 = pl.pallas_call(
        _tc_edges,
        out_shape=[jax.ShapeDtypeStruct((E // D, D), jnp.int32)] * 2,
    )(src2, dst2, et2)
    ig = ig2.reshape(E)
    ic = ic2.reshape(E)
    dst = edge_index[1]

    mesh = plsc.VectorSubcoreMesh(core_axis_name="c", subcore_axis_name="s")
    sc = pl.kernel(
        _sc_body,
        mesh=mesh,
        out_type=[jax.ShapeDtypeStruct((NC, NPAD, D), f32)] * 2,
        scratch_types=[
            pltpu.VMEM_SHARED((NPAD, D), f32),
            pltpu.VMEM_SHARED((CNT_PAD,), f32),
            pltpu.VMEM((EPT,), jnp.int32),
            pltpu.VMEM((EPT,), f32),
            pltpu.VMEM((CHUNK, D), f32),
            pltpu.VMEM((CHUNK, D), f32),
            pltpu.VMEM((CHUNK,), jnp.int32),
            pltpu.VMEM((CHUNK,), jnp.int32),
            pltpu.VMEM((CHUNK,), jnp.int32),
            pltpu.VMEM((CHUNK,), jnp.int32),
            pltpu.VMEM((CHUNK,), f32),
            pltpu.VMEM((CHUNK,), f32),
            pltpu.VMEM((640,), f32),
        ] + [pltpu.SemaphoreType.DMA] * 12,
    )
    zrow = jnp.zeros((640,), f32)
    zblk = jnp.zeros((ROWS_PT, D), f32)
    agg1p, agg2p = sc(ig, ic, dst, h1, h2, zrow, zblk)

    bs = jnp.asarray(batch_size, jnp.int32).reshape(1)
    out = pl.pallas_call(
        _tc_final,
        grid=(nb,),
        in_specs=[
            pl.BlockSpec(memory_space=pltpu.SMEM),
            pl.BlockSpec((rows, D), lambda i: (i, 0)),
            pl.BlockSpec((NC, rows, D), lambda i: (0, i, 0)),
            pl.BlockSpec((NC, rows, D), lambda i: (0, i, 0)),
            pl.BlockSpec((D, D), lambda i: (0, 0)),
            pl.BlockSpec((1, D), lambda i: (0, 0)),
            pl.BlockSpec((D, D), lambda i: (0, 0)),
            pl.BlockSpec((1, D), lambda i: (0, 0)),
            pl.BlockSpec((1, D), lambda i: (0, 0)),
        ],
        out_specs=pl.BlockSpec((rows, D), lambda i: (i, 0)),
        out_shape=jax.ShapeDtypeStruct((N, D), f32),
    )(bs, x_target, agg1p, agg2p, root1, bias1.reshape(1, D),
      root2, bias2.reshape(1, D), prelu_a.reshape(1, D))
    return out
